# baseline (speedup 1.0000x reference)
import functools

import jax
import jax.numpy as jnp
from jax import lax
from jax.experimental import pallas as pl
from jax.experimental.pallas import tpu as pltpu
from jax.experimental.pallas import tpu_sc as plsc
from jax.experimental.layout import Format, Layout

BATCH = 4096
MAXLEN = 200
EMB = 64
PADE = 128

NUM_CORES = 2
NUM_SUBCORES = 16
NUM_WORKERS = NUM_CORES * NUM_SUBCORES
NUM_CHUNKS = 4
CHUNK_B = BATCH // NUM_CHUNKS
ROWS_PER_WORKER = CHUNK_B // NUM_WORKERS
G1 = 128
G2 = MAXLEN - G1
LANES = 16
VECS_PER_ROW = EMB // LANES


def _sc_body(x_hbm, tab_hbm, pos_hbm, out_hbm, pos_v,
             idx_a, idx_b, ga, gb, oa, ob, si_a, si_b, sg_a, sg_b,
             ss_a, ss_b):
    c = lax.axis_index("c")
    s = lax.axis_index("s")
    wid = s * NUM_CORES + c
    bstart = wid * ROWS_PER_WORKER

    idx = (idx_a, idx_b)
    gbuf = (ga, gb)
    obuf = (oa, ob)
    si = (si_a, si_b)
    sg = (sg_a, sg_b)
    ss = (ss_a, ss_b)

    pltpu.sync_copy(pos_hbm, pos_v)

    def start_idx(g, p):
        pltpu.async_copy(x_hbm.at[bstart + g], idx[p], si[p])

    def wait_idx(g, p):
        pltpu.make_async_copy(x_hbm.at[bstart + g], idx[p], si[p]).wait()

    def start_gather(p):
        pltpu.async_copy(tab_hbm.at[idx[p].at[pl.ds(0, G1)]],
                         gbuf[p].at[pl.ds(0, G1)], sg[p])
        pltpu.async_copy(tab_hbm.at[idx[p].at[pl.ds(G1, G2)]],
                         gbuf[p].at[pl.ds(G1, G2)], sg[p])

    def wait_gather(p):
        pltpu.make_async_copy(tab_hbm.at[idx[p].at[pl.ds(0, G1)]],
                              gbuf[p].at[pl.ds(0, G1)], sg[p]).wait()
        pltpu.make_async_copy(tab_hbm.at[idx[p].at[pl.ds(G1, G2)]],
                              gbuf[p].at[pl.ds(G1, G2)], sg[p]).wait()

    def start_scatter(g, p):
        pltpu.async_copy(obuf[p], out_hbm.at[bstart + g], ss[p])

    def wait_scatter(g, p):
        pltpu.make_async_copy(obuf[p], out_hbm.at[bstart + g], ss[p]).wait()

    def add_pos(p):
        src = gbuf[p]
        dst = obuf[p]

        @plsc.parallel_loop(0, MAXLEN, step=1, unroll=8)
        def _row(i):
            for j in range(VECS_PER_ROW):
                sl = pl.ds(j * LANES, LANES)
                dst[i, sl] = src[i, sl] + pos_v[i, sl]

    pltpu.sync_copy(x_hbm.at[bstart], idx_a)
    start_idx(1, 1)
    start_gather(0)

    def loop_body(it, carry):
        g0 = it * 2
        for b in range(2):
            g = g0 + b
            p = b
            q = 1 - b
            wait_gather(p)
            pl.when(g + 1 < ROWS_PER_WORKER)(lambda: wait_idx(g + 1, q))
            pl.when(g + 1 < ROWS_PER_WORKER)(lambda: start_gather(q))
            pl.when(g + 2 < ROWS_PER_WORKER)(lambda: start_idx(g + 2, p))
            pl.when(g > 1)(lambda: wait_scatter(g - 2, p))
            add_pos(p)
            start_scatter(g, p)
        return carry

    lax.fori_loop(0, ROWS_PER_WORKER // 2, loop_body, 0)
    wait_scatter(ROWS_PER_WORKER - 2, 0)
    wait_scatter(ROWS_PER_WORKER - 1, 1)


def _impl(x, token_table, pos_table):
    tab_pad = jnp.pad(token_table, ((0, 0), (0, PADE - EMB)))
    mesh = plsc.VectorSubcoreMesh(core_axis_name="c", subcore_axis_name="s")
    call = pl.kernel(
        _sc_body,
        out_type=jax.ShapeDtypeStruct((CHUNK_B, MAXLEN, EMB), jnp.float32),
        mesh=mesh,
        scratch_types=[
            pltpu.VMEM((MAXLEN, EMB), jnp.float32),
            pltpu.VMEM((MAXLEN,), jnp.int32),
            pltpu.VMEM((MAXLEN,), jnp.int32),
            pltpu.VMEM((MAXLEN, PADE), jnp.float32),
            pltpu.VMEM((MAXLEN, PADE), jnp.float32),
            pltpu.VMEM((MAXLEN, EMB), jnp.float32),
            pltpu.VMEM((MAXLEN, EMB), jnp.float32),
            pltpu.SemaphoreType.DMA,
            pltpu.SemaphoreType.DMA,
            pltpu.SemaphoreType.DMA,
            pltpu.SemaphoreType.DMA,
            pltpu.SemaphoreType.DMA,
            pltpu.SemaphoreType.DMA,
        ],
        compiler_params=pltpu.CompilerParams(
            disable_bounds_checks=True,
            disable_semaphore_checks=True,
            skip_device_barrier=True,
        ),
    )
    outs = [
        call(lax.slice_in_dim(x, i * CHUNK_B, (i + 1) * CHUNK_B, axis=0),
             tab_pad, pos_table)
        for i in range(NUM_CHUNKS)
    ]
    return jnp.concatenate(outs, axis=0)


_jitted = []


def kernel(x, token_table, pos_table):
    if not _jitted:
        fmt = Format(
            Layout(major_to_minor=(0, 1, 2)),
            jax.sharding.SingleDeviceSharding(jax.devices()[0]),
        )
        _impl.__name__ = "kernel"
        _jitted.append(jax.jit(_impl, out_shardings=fmt))
    return _jitted[0](x, token_table, pos_table)


# speedup vs baseline: 1.3961x; 1.3961x over previous
import functools

import jax
import jax.numpy as jnp
from jax import lax
from jax.experimental import pallas as pl
from jax.experimental.pallas import tpu as pltpu
from jax.experimental.pallas import tpu_sc as plsc

BATCH = 4096
MAXLEN = 200
EMB = 64
PADE = 128

NUM_CORES = 2
NUM_SUBCORES = 16
NUM_WORKERS = NUM_CORES * NUM_SUBCORES
ROWS_PER_WORKER = BATCH // NUM_WORKERS
G1 = 128
G2 = MAXLEN - G1
LANES = 16
VECS_PER_ROW = EMB // LANES


def _sc_body(x_hbm, tab_hbm, pos_hbm, out_hbm, pos_v,
             idx_a, idx_b, ga, gb, oa, ob, si_a, si_b, sg_a, sg_b,
             ss_a, ss_b):
    c = lax.axis_index("c")
    s = lax.axis_index("s")
    wid = s * NUM_CORES + c
    bstart = wid * ROWS_PER_WORKER

    idx = (idx_a, idx_b)
    gbuf = (ga, gb)
    obuf = (oa, ob)
    si = (si_a, si_b)
    sg = (sg_a, sg_b)
    ss = (ss_a, ss_b)

    pltpu.sync_copy(pos_hbm, pos_v)

    def start_idx(g, p):
        pltpu.async_copy(x_hbm.at[bstart + g], idx[p], si[p])

    def wait_idx(g, p):
        pltpu.make_async_copy(x_hbm.at[bstart + g], idx[p], si[p]).wait()

    def start_gather(p):
        pltpu.async_copy(tab_hbm.at[idx[p].at[pl.ds(0, G1)]],
                         gbuf[p].at[pl.ds(0, G1)], sg[p])
        pltpu.async_copy(tab_hbm.at[idx[p].at[pl.ds(G1, G2)]],
                         gbuf[p].at[pl.ds(G1, G2)], sg[p])

    def wait_gather(p):
        pltpu.make_async_copy(tab_hbm.at[idx[p].at[pl.ds(0, G1)]],
                              gbuf[p].at[pl.ds(0, G1)], sg[p]).wait()
        pltpu.make_async_copy(tab_hbm.at[idx[p].at[pl.ds(G1, G2)]],
                              gbuf[p].at[pl.ds(G1, G2)], sg[p]).wait()

    def start_scatter(g, p):
        pltpu.async_copy(obuf[p], out_hbm.at[bstart + g], ss[p])

    def wait_scatter(g, p):
        pltpu.make_async_copy(obuf[p], out_hbm.at[bstart + g], ss[p]).wait()

    def add_pos(p):
        src = gbuf[p]
        dst = obuf[p]

        @plsc.parallel_loop(0, MAXLEN, step=1, unroll=8)
        def _row(i):
            for j in range(VECS_PER_ROW):
                sl = pl.ds(j * LANES, LANES)
                dst[i, sl] = src[i, sl] + pos_v[i, sl]

    pltpu.sync_copy(x_hbm.at[bstart], idx_a)
    start_idx(1, 1)
    start_gather(0)

    def loop_body(it, carry):
        g0 = it * 2
        for b in range(2):
            g = g0 + b
            p = b
            q = 1 - b
            wait_gather(p)
            pl.when(g + 1 < ROWS_PER_WORKER)(lambda: wait_idx(g + 1, q))
            pl.when(g + 1 < ROWS_PER_WORKER)(lambda: start_gather(q))
            pl.when(g + 2 < ROWS_PER_WORKER)(lambda: start_idx(g + 2, p))
            pl.when(g > 1)(lambda: wait_scatter(g - 2, p))
            add_pos(p)
            start_scatter(g, p)
        return carry

    lax.fori_loop(0, ROWS_PER_WORKER // 2, loop_body, 0)
    wait_scatter(ROWS_PER_WORKER - 2, 0)
    wait_scatter(ROWS_PER_WORKER - 1, 1)


@jax.jit
def kernel(x, token_table, pos_table):
    tab_pad = jnp.pad(token_table, ((0, 0), (0, PADE - EMB)))
    mesh = plsc.VectorSubcoreMesh(core_axis_name="c", subcore_axis_name="s")
    return pl.kernel(
        _sc_body,
        out_type=jax.ShapeDtypeStruct((BATCH, MAXLEN, EMB), jnp.float32),
        mesh=mesh,
        scratch_types=[
            pltpu.VMEM((MAXLEN, EMB), jnp.float32),
            pltpu.VMEM((MAXLEN,), jnp.int32),
            pltpu.VMEM((MAXLEN,), jnp.int32),
            pltpu.VMEM((MAXLEN, PADE), jnp.float32),
            pltpu.VMEM((MAXLEN, PADE), jnp.float32),
            pltpu.VMEM((MAXLEN, EMB), jnp.float32),
            pltpu.VMEM((MAXLEN, EMB), jnp.float32),
            pltpu.SemaphoreType.DMA,
            pltpu.SemaphoreType.DMA,
            pltpu.SemaphoreType.DMA,
            pltpu.SemaphoreType.DMA,
            pltpu.SemaphoreType.DMA,
            pltpu.SemaphoreType.DMA,
        ],
        compiler_params=pltpu.CompilerParams(
            disable_bounds_checks=True,
            disable_semaphore_checks=True,
            skip_device_barrier=True,
        ),
    )(x, tab_pad, pos_table)
